# Initial kernel scaffold; baseline (speedup 1.0000x reference)
import jax
import jax.numpy as jnp
from jax.experimental import pallas as pl


def kernel(x, edge_index, W1, W2, W3):
    raise NotImplementedError("write your pallas kernel here")


# speedup vs baseline: 1.4407x; 1.4407x over previous
import functools

import jax
import jax.numpy as jnp
from jax.experimental import pallas as pl
from jax.experimental.pallas import tpu as pltpu

N = 10000
E = 320000
D = 128
C = 64
K = 4
ALPHA = 0.1
BETA = 0.7
GAMMA = 0.5


def _head_body(x_ref, w3_ref, o_ref):
    h = jnp.maximum(x_ref[...], 0.0)
    o = jnp.dot(h, w3_ref[...], preferred_element_type=jnp.float32)
    m = jnp.max(o, axis=1, keepdims=True)
    lse = m + jnp.log(jnp.sum(jnp.exp(o - m), axis=1, keepdims=True))
    o_ref[...] = o - lse


def _final_head(xc, W3):
    blk = 1000
    grid = N // blk
    return pl.pallas_call(
        _head_body,
        grid=(grid,),
        in_specs=[
            pl.BlockSpec((blk, D), lambda i: (i, 0)),
            pl.BlockSpec((D, C), lambda i: (0, 0)),
        ],
        out_specs=pl.BlockSpec((blk, C), lambda i: (i, 0)),
        out_shape=jax.ShapeDtypeStruct((N, C), jnp.float32),
    )(xc, W3)


def kernel(x, edge_index, W1, W2, W3):
    row, col = edge_index[0], edge_index[1]
    h1 = jax.nn.relu(x @ W1)
    xc = h1 @ W2

    ones = jnp.ones((E,), dtype=jnp.float32)
    deg2 = jax.ops.segment_sum(ones, col, num_segments=N)
    dinv2 = jnp.where(deg2 > 0, 1.0 / jnp.sqrt(jnp.maximum(deg2, 1e-12)), 0.0)
    deg = deg2 + 1.0
    dinv = 1.0 / jnp.sqrt(deg)
    ew = dinv[row] * dinv[col]
    wloop = dinv * dinv
    ew2 = dinv2[row] * dinv2[col]

    h = xc
    for _ in range(K):
        diff = jnp.take(xc, col, axis=0) - jnp.take(xc, row, axis=0)
        d2 = jnp.sum(diff * diff, axis=1, keepdims=True)
        coef = jnp.exp(-d2 * 2.0) * 2.0
        g = jax.ops.segment_sum(ew2[:, None] * coef * diff, row, num_segments=N)
        y = xc + GAMMA * g
        s = jax.ops.segment_sum(ew[:, None] * jnp.take(y, col, axis=0), row,
                                num_segments=N)
        ay = s + wloop[:, None] * y
        xc = ALPHA * h + (1.0 - ALPHA - BETA) * xc + BETA * ay

    return _final_head(xc, W3)


# speedup vs baseline: 2.0948x; 1.4540x over previous
import functools

import jax
import jax.numpy as jnp
from jax import lax
from jax.experimental import pallas as pl
from jax.experimental.pallas import tpu as pltpu
from jax.experimental.pallas import tpu_sc as plsc

N = 10000
E = 320000
D = 128
C = 64
K = 4
ALPHA = 0.1
BETA = 0.7
GAMMA = 0.5

NC = 1
NS = 16
NW = NC * NS
NP = 10240
DUMP = N
NH = 5120
AR = 5248
ATS = AR // NS
CHB = 128
EW_CH = 158
EPW = EW_CH * CHB
EP = NW * EPW
RPW = NP // NW

f32 = jnp.float32
i32 = jnp.int32

_mesh = plsc.VectorSubcoreMesh(core_axis_name="c", subcore_axis_name="s",
                               num_cores=NC, num_subcores=NS)


def _ids():
    c = lax.axis_index("c")
    s = lax.axis_index("s")
    return c, s, s * NC + c


def _lanesum(v):
    for st in (1, 2, 4, 8):
        idx = jnp.bitwise_xor(lax.iota(i32, 16), st)
        v = v + jnp.take(v, idx)
    return v


def _bc16(r):
    return r // 8, pl.ds((r % 8) * 16, 16)


@functools.partial(
    pl.kernel,
    out_type=(jax.ShapeDtypeStruct((NP, D), f32),
              jax.ShapeDtypeStruct((NP, D), f32),
              jax.ShapeDtypeStruct((NW, EW_CH, CHB, D), f32),
              jax.ShapeDtypeStruct((AR, D), f32),
              jax.ShapeDtypeStruct((AR, D), f32)),
    mesh=_mesh,
    scratch_types=[
        pltpu.VMEM((4, CHB), i32),
        pltpu.VMEM((CHB // 8, 128), f32),
        pltpu.VMEM((8, 128), f32),
        pltpu.VMEM((8, 128), f32),
        pltpu.VMEM((CHB, D), f32),
        pltpu.VMEM((CHB, D), f32),
        pltpu.SemaphoreType.DMA,
        pltpu.VMEM_SHARED((AR, D), f32),
    ],
)
def _step_kernel(xc, xc0, rc, w2x, dinvx, dinv2x,
                 xcn, yp, gcx, gst, sst,
                 rcb, w2b, d1b, d2b, xr, xcl, sem, acc):
    c, s, w = _ids()
    arows = pl.ds(pl.multiple_of(s * ATS, 8), ATS)
    z16 = jnp.zeros((16,), f32)

    def _zero_acc():
        def zr(r, carry):
            for j in range(8):
                xcl[r, pl.ds(j * 16, 16)] = z16
            return carry

        lax.fori_loop(0, CHB, zr, 0)
        for k in range(2):
            pltpu.sync_copy(
                xcl,
                acc.at[pl.ds(pl.multiple_of(s * ATS + k * CHB, 8), CHB)])
        pltpu.sync_copy(
            xcl.at[pl.ds(0, ATS - 2 * CHB)],
            acc.at[pl.ds(pl.multiple_of(s * ATS + 2 * CHB, 8),
                         ATS - 2 * CHB)])

    _zero_acc()
    plsc.subcore_barrier()

    def ch1(chi, carry):
        pltpu.sync_copy(rc.at[w, chi], rcb)
        c1 = pltpu.async_copy(xc.at[rcb.at[3]], xr, sem)
        c2 = pltpu.async_copy(xc.at[rcb.at[2]], xcl, sem)
        c3 = pltpu.async_copy(w2x.at[w, chi], w2b, sem)
        c1.wait()
        c2.wait()
        c3.wait()

        def edge(e, carry2):
            wr, wsl = _bc16(e)
            s2 = w2b[wr, wsl]
            diffs = []
            ssq = None
            for j in range(8):
                sl = pl.ds(j * 16, 16)
                dv = xcl[e, sl] - xr[e, sl]
                diffs.append(dv)
                ssq = dv * dv if ssq is None else ssq + dv * dv
            tot = _lanesum(ssq)
            cv = jnp.exp(tot * (-2.0)) * s2
            for j in range(8):
                xcl[e, pl.ds(j * 16, 16)] = cv * diffs[j]
            return carry2

        lax.fori_loop(0, CHB, edge, carry)
        pltpu.sync_copy(xcl, acc.at[rcb.at[0]], add=True)
        pltpu.sync_copy(xcl, gcx.at[w, chi])
        return carry

    lax.fori_loop(0, EW_CH, ch1, 0)
    plsc.subcore_barrier()
    pltpu.sync_copy(acc.at[arows], gst.at[arows])
    _zero_acc()
    plsc.subcore_barrier()

    def ch2(chi, carry):
        pltpu.sync_copy(rc.at[w, chi], rcb)
        pltpu.async_copy(gcx.at[w, chi], xcl, sem).wait()
        pltpu.sync_copy(xcl, acc.at[rcb.at[1]], add=True)
        return carry

    lax.fori_loop(0, EW_CH, ch2, 0)
    plsc.subcore_barrier()

    for q in range(RPW // 64):
        base = pl.multiple_of(w * RPW + q * 64, 64)
        dbase = pl.multiple_of(w * (RPW // 8) + q * 8, 8)
        nds = pl.ds(base, 64)
        pltpu.sync_copy(xc.at[nds], xr.at[pl.ds(0, 64)])

        @pl.when(w < NH // RPW)
        def _read_g_lo():
            pltpu.sync_copy(gst.at[nds], xcl.at[pl.ds(0, 64)])

        @pl.when(w >= NH // RPW)
        def _read_g_hi():
            pltpu.sync_copy(acc.at[pl.ds(pl.multiple_of(base - NH, 64), 64)],
                            xcl.at[pl.ds(0, 64)])
        pltpu.sync_copy(dinvx.at[pl.ds(dbase, 8)], d1b)
        pltpu.sync_copy(dinv2x.at[pl.ds(dbase, 8)], d2b)

        def rb3(r, carry):
            dr, dsl = _bc16(r)
            dv = d1b[dr, dsl]
            hv = d2b[dr, dsl] * GAMMA
            for j in range(8):
                sl = pl.ds(j * 16, 16)
                xr[r, sl] = dv * (xr[r, sl] + hv * xcl[r, sl])
            return carry

        lax.fori_loop(0, 64, rb3, 0)
        pltpu.sync_copy(xr.at[pl.ds(0, 64)], yp.at[nds])
    plsc.subcore_barrier()
    _zero_acc()
    plsc.subcore_barrier()

    for half in (0, 1):
        def ch45(chi, carry, half=half):
            pltpu.sync_copy(rc.at[w, chi], rcb)
            pltpu.async_copy(yp.at[rcb.at[2]], xcl, sem).wait()
            pltpu.sync_copy(xcl, acc.at[rcb.at[half]], add=True)
            return carry

        lax.fori_loop(0, EW_CH, ch45, 0)
        plsc.subcore_barrier()
        if half == 0:
            pltpu.sync_copy(acc.at[arows], sst.at[arows])
            _zero_acc()
            plsc.subcore_barrier()

    for q in range(RPW // 64):
        base = pl.multiple_of(w * RPW + q * 64, 64)
        dbase = pl.multiple_of(w * (RPW // 8) + q * 8, 8)
        nds = pl.ds(base, 64)
        pltpu.sync_copy(xc0.at[nds], xr.at[pl.ds(64, 64)])
        pltpu.sync_copy(xc.at[nds], xr.at[pl.ds(0, 64)])
        pltpu.sync_copy(yp.at[nds], xcl.at[pl.ds(64, 64)])

        @pl.when(w < NH // RPW)
        def _read_s_lo():
            pltpu.sync_copy(sst.at[nds], xcl.at[pl.ds(0, 64)])

        @pl.when(w >= NH // RPW)
        def _read_s_hi():
            pltpu.sync_copy(acc.at[pl.ds(pl.multiple_of(base - NH, 64), 64)],
                            xcl.at[pl.ds(0, 64)])
        pltpu.sync_copy(dinvx.at[pl.ds(dbase, 8)], d1b)

        def rb6(r, carry):
            dr, dsl = _bc16(r)
            dv = d1b[dr, dsl] * BETA
            for j in range(8):
                sl = pl.ds(j * 16, 16)
                xr[r, sl] = (ALPHA * xr[64 + r, sl] +
                             (1.0 - ALPHA - BETA) * xr[r, sl] +
                             dv * (xcl[r, sl] + xcl[64 + r, sl]))
            return carry

        lax.fori_loop(0, 64, rb6, 0)
        pltpu.sync_copy(xr.at[pl.ds(0, 64)], xcn.at[nds])


def _m1_body(x_ref, w1_ref, w2_ref, o_ref):
    h1 = jnp.maximum(
        jnp.dot(x_ref[...], w1_ref[...], preferred_element_type=f32), 0.0)
    o_ref[...] = jnp.dot(h1, w2_ref[...], preferred_element_type=f32)


def _m1_call(xpad, W1, W2):
    blk = 1280
    return pl.pallas_call(
        _m1_body,
        grid=(NP // blk,),
        in_specs=[
            pl.BlockSpec((blk, D), lambda i: (i, 0)),
            pl.BlockSpec((D, D), lambda i: (0, 0)),
            pl.BlockSpec((D, D), lambda i: (0, 0)),
        ],
        out_specs=pl.BlockSpec((blk, D), lambda i: (i, 0)),
        out_shape=jax.ShapeDtypeStruct((NP, D), f32),
    )(xpad, W1, W2)


def _head_body(x_ref, w3_ref, o_ref):
    h = jnp.maximum(x_ref[...], 0.0)
    o = jnp.dot(h, w3_ref[...], preferred_element_type=f32)
    m = jnp.max(o, axis=1, keepdims=True)
    lse = m + jnp.log(jnp.sum(jnp.exp(o - m), axis=1, keepdims=True))
    o_ref[...] = o - lse


def _head_call(xc, W3):
    blk = 1280
    return pl.pallas_call(
        _head_body,
        grid=(NP // blk,),
        in_specs=[
            pl.BlockSpec((blk, D), lambda i: (i, 0)),
            pl.BlockSpec((D, C), lambda i: (0, 0)),
        ],
        out_specs=pl.BlockSpec((blk, C), lambda i: (i, 0)),
        out_shape=jax.ShapeDtypeStruct((NP, C), f32),
    )(xc, W3)


def _expand16(v):
    return jnp.repeat(v.reshape(-1, 8), 16, axis=1)


def kernel(x, edge_index, W1, W2, W3):
    row = edge_index[0]
    col = edge_index[1]
    pad_e = EP - E
    rowp = jnp.concatenate([row, jnp.full((pad_e,), DUMP, i32)])
    colp = jnp.concatenate([col, jnp.full((pad_e,), DUMP, i32)])
    rowlo = jnp.where(rowp < NH, rowp, NH)
    rowhi = jnp.where(rowp >= NH, rowp - NH, NH)
    rc = jnp.stack([rowlo, rowhi, colp, rowp],
                   axis=0).reshape(4, NW, EW_CH, CHB).transpose(1, 2, 0, 3)
    xpad = jnp.zeros((NP, D), f32).at[:N].set(x)

    deg2 = jax.ops.segment_sum(jnp.ones((E,), f32), col, num_segments=NP)
    dinv2 = jnp.where(deg2 > 0, 1.0 / jnp.sqrt(jnp.maximum(deg2, 1e-12)), 0.0)
    dinv = 1.0 / jnp.sqrt(deg2 + 1.0)
    w2e = 2.0 * dinv2[colp]
    w2x = _expand16(w2e).reshape(NW, EW_CH, CHB // 8, 128)
    dinvx = _expand16(dinv)
    dinv2x = _expand16(dinv2)

    xc = _m1_call(xpad, W1, W2)
    xc0 = xc
    for _ in range(K):
        xc, _yp, _gcx, _gst, _sst = _step_kernel(
            xc, xc0, rc, w2x, dinvx, dinv2x)

    out = _head_call(xc, W3)
    return out[:N]
